# speedup vs baseline: 5.7161x; 5.7161x over previous
import functools

import jax
import jax.numpy as jnp
from jax import lax
from jax.experimental import pallas as pl
from jax.experimental.pallas import tpu as pltpu
from jax.experimental.pallas import tpu_sc as plsc

EMB_DIM = 512
POS_DIM = 25
ENT_DIM = 512

B_TOTAL = 16 * 128 * 30
NW = 32
B_PER_W = B_TOTAL // NW
CH = 128
N_CHUNK = B_PER_W // CH

BLK = 512
NBLK = B_TOTAL // BLK


def _make_sc_gather():
    mesh = plsc.VectorSubcoreMesh(core_axis_name="c", subcore_axis_name="s")

    @functools.partial(
        pl.kernel,
        mesh=mesh,
        out_type=(
            jax.ShapeDtypeStruct((B_TOTAL, EMB_DIM), jnp.float32),
            jax.ShapeDtypeStruct((B_TOTAL,), jnp.int32),
        ),
        scratch_types=[
            pltpu.VMEM((CH,), jnp.int32),
            pltpu.VMEM((CH, EMB_DIM), jnp.float32),
            pltpu.VMEM((CH,), jnp.int32),
            pltpu.SemaphoreType.DMA,
            pltpu.SemaphoreType.DMA,
        ],
    )
    def sc_gather(table_hbm, pidx_hbm, ids_hbm, g_hbm, pid_hbm,
                  idx_v, rows_v, pid_v, sem_r, sem_p):
        nc = 2
        wid = lax.axis_index("s") * nc + lax.axis_index("c")

        def body(i, carry):
            base = wid * B_PER_W + i * CH
            pltpu.sync_copy(ids_hbm.at[pl.ds(base, CH)], idx_v)
            cp_r = pltpu.async_copy(table_hbm.at[idx_v], rows_v, sem_r)
            cp_p = pltpu.async_copy(pidx_hbm.at[idx_v], pid_v, sem_p)
            cp_r.wait()
            cp_p.wait()
            pltpu.sync_copy(rows_v, g_hbm.at[pl.ds(base, CH)])
            pltpu.sync_copy(pid_v, pid_hbm.at[pl.ds(base, CH)])
            return carry

        lax.fori_loop(0, N_CHUNK, body, 0)

    return sc_gather


_sc_gather = _make_sc_gather()


def _tc_body(g_ref, w_ref, pos_ref, b_ref, pid_ref, out_ref):
    g = g_ref[...]
    we = w_ref[:, :EMB_DIM]
    wp = w_ref[:, EMB_DIM:]
    pos16 = pos_ref[...]
    bias16 = lax.dot_general(
        pos16, wp, (((1,), (1,)), ((), ())),
        preferred_element_type=jnp.float32) + b_ref[...]
    pid = pid_ref[...]
    onehot = (pid == lax.broadcasted_iota(jnp.int32, (BLK, 16), 1)
              ).astype(jnp.float32)
    out = lax.dot_general(
        g, we, (((1,), (1,)), ((), ())),
        preferred_element_type=jnp.float32)
    out = out + lax.dot_general(
        onehot, bias16, (((1,), (0,)), ((), ())),
        preferred_element_type=jnp.float32)
    out_ref[...] = out


def _tc_project(g, w, pos16, b2, pid2):
    return pl.pallas_call(
        _tc_body,
        grid=(NBLK,),
        in_specs=[
            pl.BlockSpec((BLK, EMB_DIM), lambda i: (i, 0)),
            pl.BlockSpec((ENT_DIM, EMB_DIM + POS_DIM), lambda i: (0, 0)),
            pl.BlockSpec((16, POS_DIM), lambda i: (0, 0)),
            pl.BlockSpec((1, ENT_DIM), lambda i: (0, 0)),
            pl.BlockSpec((BLK, 1), lambda i: (i, 0)),
        ],
        out_specs=pl.BlockSpec((BLK, ENT_DIM), lambda i: (i, 0)),
        out_shape=jax.ShapeDtypeStruct((B_TOTAL, ENT_DIM), jnp.float32),
    )(g, w, pos16, b2, pid2)


def kernel(entity_ids, entity_table, pos_table, entity_id_to_pos_index, W, b):
    shape = entity_ids.shape
    ids = entity_ids.reshape(-1).astype(jnp.int32)
    pidx = entity_id_to_pos_index.astype(jnp.int32)
    g, pid = _sc_gather(entity_table, pidx, ids)
    out = _tc_project(g, W, pos_table[:16], b.reshape(1, ENT_DIM),
                      pid.reshape(B_TOTAL, 1))
    return out.reshape(*shape, ENT_DIM)


# speedup vs baseline: 8.1054x; 1.4180x over previous
import functools

import jax
import jax.numpy as jnp
from jax import lax
from jax.experimental import pallas as pl
from jax.experimental.pallas import tpu as pltpu
from jax.experimental.pallas import tpu_sc as plsc

EMB_DIM = 512
POS_DIM = 25
ENT_DIM = 512

BATCH = 16
NCAND = 128
NENT = 30

B_TOTAL = BATCH * NCAND * NENT
NW = 32
B_PER_W = B_TOTAL // NW
CH = 96
N_CHUNK = B_PER_W // CH

CBLK = 32
GRID = (BATCH, NCAND // CBLK)
RBLK = CBLK * NENT


def _make_sc_gather():
    mesh = plsc.VectorSubcoreMesh(core_axis_name="c", subcore_axis_name="s")

    @functools.partial(
        pl.kernel,
        mesh=mesh,
        out_type=(
            jax.ShapeDtypeStruct((B_TOTAL, EMB_DIM), jnp.float32),
            jax.ShapeDtypeStruct((B_TOTAL,), jnp.int32),
        ),
        scratch_types=[
            pltpu.VMEM((B_PER_W,), jnp.int32),
            pltpu.VMEM((B_PER_W,), jnp.int32),
            pltpu.VMEM((CH, EMB_DIM), jnp.float32),
            pltpu.VMEM((CH, EMB_DIM), jnp.float32),
            pltpu.SemaphoreType.DMA,
            pltpu.SemaphoreType.DMA,
            pltpu.SemaphoreType.DMA,
            pltpu.SemaphoreType.DMA,
            pltpu.SemaphoreType.DMA,
        ],
    )
    def sc_gather(table_hbm, pidx_hbm, ids_hbm, g_hbm, pid_hbm,
                  idx_all, pid_all, rows0, rows1,
                  gsem0, gsem1, wsem0, wsem1, psem):
        nc = 2
        wid = lax.axis_index("s") * nc + lax.axis_index("c")
        base = wid * B_PER_W
        rows = (rows0, rows1)
        gsem = (gsem0, gsem1)
        wsem = (wsem0, wsem1)

        pltpu.sync_copy(ids_hbm.at[pl.ds(base, B_PER_W)], idx_all)
        pid_cp = pltpu.async_copy(pidx_hbm.at[idx_all], pid_all, psem)

        def g_issue(c, b):
            return pltpu.async_copy(
                table_hbm.at[idx_all.at[pl.ds(c * CH, CH)]], rows[b], gsem[b])

        def g_wait(b):
            pltpu.make_async_copy(
                table_hbm.at[idx_all.at[pl.ds(0, CH)]], rows[b], gsem[b]
            ).wait()

        def w_issue(c, b):
            return pltpu.async_copy(
                rows[b], g_hbm.at[pl.ds(base + c * CH, CH)], wsem[b])

        def w_wait(b):
            pltpu.make_async_copy(
                rows[b], g_hbm.at[pl.ds(base, CH)], wsem[b]).wait()

        g_issue(0, 0)

        def body(c2, carry):
            for b in range(2):
                c = c2 * 2 + b
                nb = 1 - b

                @pl.when(c + 1 < N_CHUNK)
                def _():
                    @pl.when(c >= 1)
                    def _():
                        w_wait(nb)

                    g_issue(c + 1, nb)

                g_wait(b)
                w_issue(c, b)
            return carry

        lax.fori_loop(0, N_CHUNK // 2, body, 0)
        w_wait(0)
        w_wait(1)
        pid_cp.wait()
        pltpu.sync_copy(pid_all, pid_hbm.at[pl.ds(base, B_PER_W)])

    return sc_gather


_sc_gather = _make_sc_gather()


def _tc_body(g_ref, w_ref, pos_ref, b_ref, pid_ref, out_ref):
    g = g_ref[...]
    we = w_ref[:, :EMB_DIM]
    wp = w_ref[:, EMB_DIM:]
    pos16 = pos_ref[...]
    bias16 = lax.dot_general(
        pos16, wp, (((1,), (1,)), ((), ())),
        preferred_element_type=jnp.float32) + b_ref[...]
    pid = pid_ref[...]
    onehot = (pid == lax.broadcasted_iota(jnp.int32, (RBLK, 16), 1)
              ).astype(jnp.float32)
    out = lax.dot_general(
        g, we, (((1,), (1,)), ((), ())),
        preferred_element_type=jnp.float32)
    out = out + lax.dot_general(
        onehot, bias16, (((1,), (0,)), ((), ())),
        preferred_element_type=jnp.float32)
    out_ref[...] = out.reshape(1, CBLK, NENT, ENT_DIM)


def _tc_project(g, w, pos16, b2, pid2):
    return pl.pallas_call(
        _tc_body,
        grid=GRID,
        in_specs=[
            pl.BlockSpec((RBLK, EMB_DIM), lambda i, j: (i * GRID[1] + j, 0)),
            pl.BlockSpec((ENT_DIM, EMB_DIM + POS_DIM), lambda i, j: (0, 0)),
            pl.BlockSpec((16, POS_DIM), lambda i, j: (0, 0)),
            pl.BlockSpec((1, ENT_DIM), lambda i, j: (0, 0)),
            pl.BlockSpec((RBLK, 1), lambda i, j: (i * GRID[1] + j, 0)),
        ],
        out_specs=pl.BlockSpec((1, CBLK, NENT, ENT_DIM),
                               lambda i, j: (i, j, 0, 0)),
        out_shape=jax.ShapeDtypeStruct((BATCH, NCAND, NENT, ENT_DIM),
                                       jnp.float32),
    )(g, w, pos16, b2, pid2)


def kernel(entity_ids, entity_table, pos_table, entity_id_to_pos_index, W, b):
    ids = entity_ids.reshape(-1).astype(jnp.int32)
    pidx = entity_id_to_pos_index.astype(jnp.int32)
    g, pid = _sc_gather(entity_table, pidx, ids)
    return _tc_project(g, W, pos_table[:16], b.reshape(1, ENT_DIM),
                       pid.reshape(B_TOTAL, 1))


# speedup vs baseline: 13.0462x; 1.6096x over previous
import functools

import jax
import jax.numpy as jnp
from jax import lax
from jax.experimental import pallas as pl
from jax.experimental.pallas import tpu as pltpu
from jax.experimental.pallas import tpu_sc as plsc

EMB_DIM = 512
POS_DIM = 25
ENT_DIM = 512

BATCH = 16
NCAND = 128
NENT = 30

B_TOTAL = BATCH * NCAND * NENT
NW = 32
B_PER_W = B_TOTAL // NW
CH = 96
N_CHUNK = B_PER_W // CH

RBLK = NENT * NCAND


def _make_sc_gather():
    mesh = plsc.VectorSubcoreMesh(core_axis_name="c", subcore_axis_name="s")

    @functools.partial(
        pl.kernel,
        mesh=mesh,
        out_type=(
            jax.ShapeDtypeStruct((B_TOTAL, EMB_DIM), jnp.float32),
            jax.ShapeDtypeStruct((B_TOTAL,), jnp.int32),
        ),
        scratch_types=[
            pltpu.VMEM((B_PER_W,), jnp.int32),
            pltpu.VMEM((B_PER_W,), jnp.int32),
            pltpu.VMEM((CH, EMB_DIM), jnp.float32),
            pltpu.VMEM((CH, EMB_DIM), jnp.float32),
            pltpu.SemaphoreType.DMA,
            pltpu.SemaphoreType.DMA,
            pltpu.SemaphoreType.DMA,
            pltpu.SemaphoreType.DMA,
            pltpu.SemaphoreType.DMA,
        ],
    )
    def sc_gather(table_hbm, pidx_hbm, ids_hbm, g_hbm, pid_hbm,
                  idx_all, pid_all, rows0, rows1,
                  gsem0, gsem1, wsem0, wsem1, psem):
        nc = 2
        wid = lax.axis_index("s") * nc + lax.axis_index("c")
        base = wid * B_PER_W
        rows = (rows0, rows1)
        gsem = (gsem0, gsem1)
        wsem = (wsem0, wsem1)

        pltpu.sync_copy(ids_hbm.at[pl.ds(base, B_PER_W)], idx_all)
        pid_cp = pltpu.async_copy(pidx_hbm.at[idx_all], pid_all, psem)

        def g_issue(c, b):
            return pltpu.async_copy(
                table_hbm.at[idx_all.at[pl.ds(c * CH, CH)]], rows[b], gsem[b])

        def g_wait(b):
            pltpu.make_async_copy(
                table_hbm.at[idx_all.at[pl.ds(0, CH)]], rows[b], gsem[b]
            ).wait()

        def w_issue(c, b):
            return pltpu.async_copy(
                rows[b], g_hbm.at[pl.ds(base + c * CH, CH)], wsem[b])

        def w_wait(b):
            pltpu.make_async_copy(
                rows[b], g_hbm.at[pl.ds(base, CH)], wsem[b]).wait()

        g_issue(0, 0)

        def body(c2, carry):
            for b in range(2):
                c = c2 * 2 + b
                nb = 1 - b

                @pl.when(c + 1 < N_CHUNK)
                def _():
                    @pl.when(c >= 1)
                    def _():
                        w_wait(nb)

                    g_issue(c + 1, nb)

                g_wait(b)
                w_issue(c, b)
            return carry

        lax.fori_loop(0, N_CHUNK // 2, body, 0)
        w_wait(0)
        w_wait(1)
        pid_cp.wait()
        pltpu.sync_copy(pid_all, pid_hbm.at[pl.ds(base, B_PER_W)])

    return sc_gather


_sc_gather = _make_sc_gather()


def _tc_body(g_ref, w_ref, pos_ref, b_ref, pid_ref, out_ref):
    g = g_ref[...].reshape(RBLK, EMB_DIM)
    we = w_ref[:, :EMB_DIM]
    wp = w_ref[:, EMB_DIM:]
    pos16 = pos_ref[...]
    bias16 = lax.dot_general(
        pos16, wp, (((1,), (1,)), ((), ())),
        preferred_element_type=jnp.float32) + b_ref[...]
    pid = pid_ref[...]
    onehot = (pid == lax.broadcasted_iota(jnp.int32, (RBLK, 16), 1)
              ).astype(jnp.float32)
    out = lax.dot_general(
        g, we, (((1,), (1,)), ((), ())),
        preferred_element_type=jnp.float32)
    out = out + lax.dot_general(
        onehot, bias16, (((1,), (0,)), ((), ())),
        preferred_element_type=jnp.float32)
    out_ref[...] = out.reshape(1, NENT, NCAND, ENT_DIM)


def _tc_project(g3, w, pos16, b2, pid2):
    return pl.pallas_call(
        _tc_body,
        grid=(BATCH,),
        in_specs=[
            pl.BlockSpec((NENT, NCAND, EMB_DIM), lambda i: (i, 0, 0)),
            pl.BlockSpec((ENT_DIM, EMB_DIM + POS_DIM), lambda i: (0, 0)),
            pl.BlockSpec((16, POS_DIM), lambda i: (0, 0)),
            pl.BlockSpec((1, ENT_DIM), lambda i: (0, 0)),
            pl.BlockSpec((RBLK, 1), lambda i: (i, 0)),
        ],
        out_specs=pl.BlockSpec((1, NENT, NCAND, ENT_DIM),
                               lambda i: (i, 0, 0, 0)),
        out_shape=jax.ShapeDtypeStruct((BATCH, NENT, NCAND, ENT_DIM),
                                       jnp.float32),
    )(g3, w, pos16, b2, pid2)


def kernel(entity_ids, entity_table, pos_table, entity_id_to_pos_index, W, b):
    ids = entity_ids.transpose(0, 2, 1).reshape(-1).astype(jnp.int32)
    pidx = entity_id_to_pos_index.astype(jnp.int32)
    g, pid = _sc_gather(entity_table, pidx, ids)
    g3 = g.reshape(BATCH * NENT, NCAND, EMB_DIM)
    out_t = _tc_project(g3, W, pos_table[:16], b.reshape(1, ENT_DIM),
                        pid.reshape(B_TOTAL, 1))
    return out_t.transpose(0, 2, 1, 3)
